# baseline (speedup 1.0000x reference)
import functools

import jax
import jax.numpy as jnp
from jax import lax
from jax.experimental import pallas as pl
from jax.experimental.pallas import tpu as pltpu
from jax.experimental.pallas import tpu_sc as plsc

B = 4096
NC = 100
NH = 64
NL = 10
CH = NC * NH
LANES = 16
NW = 32
B_SC = 1536
ROWS_PER_W = B_SC // NW
CHUNKS = CH // LANES
HCH = NH // LANES
BB = 256
NBLK_TC = (B - B_SC) // BB


def _sc_partials_kernel(z_hbm, labels_hbm, a_out, q_out, c_out,
                        a_v, zbuf0, zbuf1, labels_v, q_v, c_v, sem0, sem1):
    nc_axis = 2
    wid = lax.axis_index("s") * nc_axis + lax.axis_index("c")
    base = wid * ROWS_PER_W

    zeros = jnp.zeros((LANES,), jnp.float32)
    ones = jnp.ones((LANES,), jnp.float32)

    pltpu.sync_copy(labels_hbm.at[pl.ds(base, ROWS_PER_W)], labels_v)

    def zero_body(i, c):
        for u in range(8):
            a_v[pl.ds(8 * LANES * i + LANES * u, LANES)] = zeros
        return c
    lax.fori_loop(0, NL * CHUNKS // 8, zero_body, 0)
    for l in range(NL):
        q_v[pl.ds(LANES * l, LANES)] = zeros
        c_v[pl.ds(LANES * l, LANES)] = zeros

    bufs = (zbuf0, zbuf1)
    sems = (sem0, sem1)

    def start(k, row):
        pltpu.make_async_copy(z_hbm.at[base + row], bufs[k], sems[k]).start()

    def wait(k):
        pltpu.make_async_copy(z_hbm.at[base], bufs[k], sems[k]).wait()

    UNR = 16
    NACC = 8

    def process(buf, lab):
        off = lab * CH

        def body(j, qs):
            o = UNR * LANES * j
            zv = [buf[pl.ds(o + LANES * u, LANES)] for u in range(UNR)]
            for u in range(UNR):
                plsc.addupdate(a_v.at[pl.ds(off + o + LANES * u, LANES)],
                               zv[u])
            qs = list(qs)
            for u in range(UNR):
                qs[u % NACC] = qs[u % NACC] + zv[u] * zv[u]
            return tuple(qs)
        qs = lax.fori_loop(0, CHUNKS // UNR, body, (zeros,) * NACC)
        q = ((qs[0] + qs[1]) + (qs[2] + qs[3])) + \
            ((qs[4] + qs[5]) + (qs[6] + qs[7]))
        plsc.addupdate(q_v.at[pl.ds(lab * LANES, LANES)], q)
        plsc.addupdate(c_v.at[pl.ds(lab * LANES, LANES)], ones)

    NGROUPS = ROWS_PER_W // LANES
    start(0, 0)
    start(1, 1)

    def group_body(g, c):
        lv = labels_v[pl.ds(LANES * g, LANES)]
        for u in range(LANES):
            k = u % 2
            wait(k)
            process(bufs[k], lv[u])
            start(k, LANES * g + u + 2)
        return c
    lax.fori_loop(0, NGROUPS - 1, group_body, 0)
    lv = labels_v[pl.ds(LANES * (NGROUPS - 1), LANES)]
    for u in range(LANES):
        k = u % 2
        wait(k)
        process(bufs[k], lv[u])
        if u < LANES - 2:
            start(k, LANES * (NGROUPS - 1) + u + 2)

    pltpu.sync_copy(a_v, a_out.at[wid])
    pltpu.sync_copy(q_v, q_out.at[wid])
    pltpu.sync_copy(c_v, c_out.at[wid])


def _tc_partials_body(z_ref, lab_ref, a_ref, q_ref, c_ref):
    i = pl.program_id(0)
    z = z_ref[...].reshape(BB, CH)
    lab = lab_ref[0]
    oh = (lab[:, None] == lax.broadcasted_iota(jnp.int32, (1, LANES), 1)
          ).astype(jnp.float32)
    a = lax.dot_general(oh, z, (((0,), (0,)), ((), ())),
                        preferred_element_type=jnp.float32)
    rs = jnp.sum(z * z, axis=1, keepdims=True)
    ql = jnp.sum(rs * oh, axis=0, keepdims=True)
    cl = jnp.sum(oh, axis=0, keepdims=True)

    @pl.when(i == 0)
    def _():
        a_ref[...] = jnp.zeros_like(a_ref)
        q_ref[...] = jnp.zeros_like(q_ref)
        c_ref[...] = jnp.zeros_like(c_ref)
    a_ref[...] += a
    q_ref[...] += jnp.broadcast_to(ql, (8, LANES))
    c_ref[...] += jnp.broadcast_to(cl, (8, LANES))


def _finalize_body(a_sc, q_sc, c_sc, a_tc, q_tc, c_tc, out_ref):
    a = jnp.sum(a_sc[...], axis=0) + a_tc[...][:NL, :]
    q = jnp.sum(q_sc[...], axis=(0, 2)) + q_tc[...][0, :NL]
    n = jnp.sum(c_sc[...][:, :, 0], axis=0) + c_tc[...][0, :NL]
    ssq = jnp.sum(a * a, axis=1)
    safe = jnp.maximum(n, 1.0)
    chf = jnp.float32(CH)
    eps2 = jnp.float32(1e-16)
    mse = q / (safe * chf) - ssq / (safe * safe * chf) + eps2
    out_ref[...] = jnp.sum(jnp.where(n > 0, mse, 0.0)).reshape(1, 1)


@jax.jit
def _run(z3, labels):
    mesh = plsc.VectorSubcoreMesh(core_axis_name="c", subcore_axis_name="s")
    sc = pl.kernel(
        _sc_partials_kernel,
        mesh=mesh,
        out_type=(
            jax.ShapeDtypeStruct((NW, NL * CH), jnp.float32),
            jax.ShapeDtypeStruct((NW, NL * LANES), jnp.float32),
            jax.ShapeDtypeStruct((NW, NL * LANES), jnp.float32),
        ),
        scratch_types=[
            pltpu.VMEM((NL * CH,), jnp.float32),
            pltpu.VMEM((CH,), jnp.float32),
            pltpu.VMEM((CH,), jnp.float32),
            pltpu.VMEM((ROWS_PER_W,), jnp.int32),
            pltpu.VMEM((NL * LANES,), jnp.float32),
            pltpu.VMEM((NL * LANES,), jnp.float32),
            pltpu.SemaphoreType.DMA,
            pltpu.SemaphoreType.DMA,
        ],
    )
    z_sc = z3[:B_SC].reshape(B_SC, CH)
    a_sc, q_sc, c_sc = sc(z_sc, labels)

    a_tc, q_tc, c_tc = pl.pallas_call(
        _tc_partials_body,
        grid=(NBLK_TC,),
        in_specs=[
            pl.BlockSpec((BB, NC, NH), lambda i: (B_SC // BB + i, 0, 0)),
            pl.BlockSpec((1, BB), lambda i: (0, B_SC // BB + i)),
        ],
        out_specs=[
            pl.BlockSpec((LANES, CH), lambda i: (0, 0)),
            pl.BlockSpec((8, LANES), lambda i: (0, 0)),
            pl.BlockSpec((8, LANES), lambda i: (0, 0)),
        ],
        out_shape=[
            jax.ShapeDtypeStruct((LANES, CH), jnp.float32),
            jax.ShapeDtypeStruct((8, LANES), jnp.float32),
            jax.ShapeDtypeStruct((8, LANES), jnp.float32),
        ],
    )(z3, labels.reshape(1, B))

    out = pl.pallas_call(
        _finalize_body,
        out_shape=jax.ShapeDtypeStruct((1, 1), jnp.float32),
    )(a_sc.reshape(NW, NL, CH),
      q_sc.reshape(NW, NL, LANES),
      c_sc.reshape(NW, NL, LANES),
      a_tc, q_tc, c_tc)
    return out[0, 0]


def kernel(z, labels):
    return _run(z, labels)


# speedup vs baseline: 1.0574x; 1.0574x over previous
import functools

import jax
import jax.numpy as jnp
from jax import lax
from jax.experimental import pallas as pl
from jax.experimental.pallas import tpu as pltpu
from jax.experimental.pallas import tpu_sc as plsc

B = 4096
NC = 100
NH = 64
NL = 10
CH = NC * NH
LANES = 16
NW = 32
B_SC = 1536
ROWS_PER_W = B_SC // NW
CHUNKS = CH // LANES
HCH = NH // LANES
BB = 256
NBLK_TC = (B - B_SC) // BB


def _sc_partials_kernel(z_hbm, labels_hbm, a_out, q_out, c_out,
                        a_v, zbuf0, zbuf1, labels_v, q_v, c_v, sem0, sem1):
    nc_axis = 2
    wid = lax.axis_index("s") * nc_axis + lax.axis_index("c")
    base = wid * ROWS_PER_W

    zeros = jnp.zeros((LANES,), jnp.float32)
    ones = jnp.ones((LANES,), jnp.float32)

    pltpu.sync_copy(labels_hbm.at[pl.ds(base, ROWS_PER_W)], labels_v)

    def zero_body(i, c):
        for u in range(8):
            a_v[pl.ds(8 * LANES * i + LANES * u, LANES)] = zeros
        return c
    lax.fori_loop(0, NL * CHUNKS // 8, zero_body, 0)
    for l in range(NL):
        q_v[pl.ds(LANES * l, LANES)] = zeros
        c_v[pl.ds(LANES * l, LANES)] = zeros

    bufs = (zbuf0, zbuf1)
    sems = (sem0, sem1)

    def start(k, row):
        pltpu.make_async_copy(z_hbm.at[base + row], bufs[k], sems[k]).start()

    def wait(k):
        pltpu.make_async_copy(z_hbm.at[base], bufs[k], sems[k]).wait()

    UNR = 16
    NACC = 8

    def process(buf, lab):
        off = lab * CH

        def body(j, qs):
            o = UNR * LANES * j
            zv = [buf[pl.ds(o + LANES * u, LANES)] for u in range(UNR)]
            for u in range(UNR):
                plsc.addupdate(a_v.at[pl.ds(off + o + LANES * u, LANES)],
                               zv[u])
            qs = list(qs)
            for u in range(UNR):
                qs[u % NACC] = qs[u % NACC] + zv[u] * zv[u]
            return tuple(qs)
        qs = lax.fori_loop(0, CHUNKS // UNR, body, (zeros,) * NACC)
        q = ((qs[0] + qs[1]) + (qs[2] + qs[3])) + \
            ((qs[4] + qs[5]) + (qs[6] + qs[7]))
        plsc.addupdate(q_v.at[pl.ds(lab * LANES, LANES)], q)
        plsc.addupdate(c_v.at[pl.ds(lab * LANES, LANES)], ones)

    NGROUPS = ROWS_PER_W // LANES
    start(0, 0)
    start(1, 1)

    def group_body(g, c):
        lv = labels_v[pl.ds(LANES * g, LANES)]
        for u in range(LANES):
            k = u % 2
            wait(k)
            process(bufs[k], lv[u])
            start(k, LANES * g + u + 2)
        return c
    lax.fori_loop(0, NGROUPS - 1, group_body, 0)
    lv = labels_v[pl.ds(LANES * (NGROUPS - 1), LANES)]
    for u in range(LANES):
        k = u % 2
        wait(k)
        process(bufs[k], lv[u])
        if u < LANES - 2:
            start(k, LANES * (NGROUPS - 1) + u + 2)

    pltpu.sync_copy(a_v, a_out.at[wid])
    pltpu.sync_copy(q_v, q_out.at[wid])
    pltpu.sync_copy(c_v, c_out.at[wid])


def _relayout_body(z_ref, out_ref):
    out_ref[...] = z_ref[...].reshape(BB, CH)


def _tc_partials_body(z_ref, lab_ref, a_ref, q_ref, c_ref):
    i = pl.program_id(0)
    z = z_ref[...].reshape(BB, CH)
    lab = lab_ref[0]
    oh = (lab[:, None] == lax.broadcasted_iota(jnp.int32, (1, LANES), 1)
          ).astype(jnp.float32)
    a = lax.dot_general(oh, z, (((0,), (0,)), ((), ())),
                        preferred_element_type=jnp.float32)
    rs = jnp.sum(z * z, axis=1, keepdims=True)
    ql = jnp.sum(rs * oh, axis=0, keepdims=True)
    cl = jnp.sum(oh, axis=0, keepdims=True)

    @pl.when(i == 0)
    def _():
        a_ref[...] = jnp.zeros_like(a_ref)
        q_ref[...] = jnp.zeros_like(q_ref)
        c_ref[...] = jnp.zeros_like(c_ref)
    a_ref[...] += a
    q_ref[...] += jnp.broadcast_to(ql, (8, LANES))
    c_ref[...] += jnp.broadcast_to(cl, (8, LANES))


def _finalize_body(a_sc, q_sc, c_sc, a_tc, q_tc, c_tc, out_ref):
    a = jnp.sum(a_sc[...], axis=0) + a_tc[...][:NL, :]
    q = jnp.sum(q_sc[...], axis=(0, 2)) + q_tc[...][0, :NL]
    n = jnp.sum(c_sc[...][:, :, 0], axis=0) + c_tc[...][0, :NL]
    ssq = jnp.sum(a * a, axis=1)
    safe = jnp.maximum(n, 1.0)
    chf = jnp.float32(CH)
    eps2 = jnp.float32(1e-16)
    mse = q / (safe * chf) - ssq / (safe * safe * chf) + eps2
    out_ref[...] = jnp.sum(jnp.where(n > 0, mse, 0.0)).reshape(1, 1)


@jax.jit
def _run(z3, labels):
    mesh = plsc.VectorSubcoreMesh(core_axis_name="c", subcore_axis_name="s")
    sc = pl.kernel(
        _sc_partials_kernel,
        mesh=mesh,
        out_type=(
            jax.ShapeDtypeStruct((NW, NL * CH), jnp.float32),
            jax.ShapeDtypeStruct((NW, NL * LANES), jnp.float32),
            jax.ShapeDtypeStruct((NW, NL * LANES), jnp.float32),
        ),
        scratch_types=[
            pltpu.VMEM((NL * CH,), jnp.float32),
            pltpu.VMEM((CH,), jnp.float32),
            pltpu.VMEM((CH,), jnp.float32),
            pltpu.VMEM((ROWS_PER_W,), jnp.int32),
            pltpu.VMEM((NL * LANES,), jnp.float32),
            pltpu.VMEM((NL * LANES,), jnp.float32),
            pltpu.SemaphoreType.DMA,
            pltpu.SemaphoreType.DMA,
        ],
    )
    z_sc = pl.pallas_call(
        _relayout_body,
        grid=(B_SC // BB,),
        in_specs=[pl.BlockSpec((BB, NC, NH), lambda i: (i, 0, 0))],
        out_specs=pl.BlockSpec((BB, CH), lambda i: (i, 0)),
        out_shape=jax.ShapeDtypeStruct((B_SC, CH), jnp.float32),
    )(z3)
    a_sc, q_sc, c_sc = sc(z_sc, labels)

    a_tc, q_tc, c_tc = pl.pallas_call(
        _tc_partials_body,
        grid=(NBLK_TC,),
        in_specs=[
            pl.BlockSpec((BB, NC, NH), lambda i: (B_SC // BB + i, 0, 0)),
            pl.BlockSpec((1, BB), lambda i: (0, B_SC // BB + i)),
        ],
        out_specs=[
            pl.BlockSpec((LANES, CH), lambda i: (0, 0)),
            pl.BlockSpec((8, LANES), lambda i: (0, 0)),
            pl.BlockSpec((8, LANES), lambda i: (0, 0)),
        ],
        out_shape=[
            jax.ShapeDtypeStruct((LANES, CH), jnp.float32),
            jax.ShapeDtypeStruct((8, LANES), jnp.float32),
            jax.ShapeDtypeStruct((8, LANES), jnp.float32),
        ],
    )(z3, labels.reshape(1, B))

    out = pl.pallas_call(
        _finalize_body,
        out_shape=jax.ShapeDtypeStruct((1, 1), jnp.float32),
    )(a_sc.reshape(NW, NL, CH),
      q_sc.reshape(NW, NL, LANES),
      c_sc.reshape(NW, NL, LANES),
      a_tc, q_tc, c_tc)
    return out[0, 0]


def kernel(z, labels):
    return _run(z, labels)


# speedup vs baseline: 1.6689x; 1.5784x over previous
import functools

import jax
import jax.numpy as jnp
from jax import lax
from jax.experimental import pallas as pl
from jax.experimental.pallas import tpu as pltpu
from jax.experimental.pallas import tpu_sc as plsc

B = 4096
NC = 100
NH = 64
NL = 10
CH = NC * NH
LANES = 16
NW = 32
B_SC = 1536
ROWS_PER_W = B_SC // NW
CHUNKS = CH // LANES
HCH = NH // LANES
BB = 256
NBLK_TC = (B - B_SC) // BB


def _sc_partials_kernel(z_hbm, labels_hbm, a_out, q_out, c_out,
                        a_v, zbuf0, zbuf1, labels_v, q_v, c_v, sem0, sem1):
    nc_axis = 2
    wid = lax.axis_index("s") * nc_axis + lax.axis_index("c")
    base = wid * ROWS_PER_W

    zeros = jnp.zeros((LANES,), jnp.float32)
    ones = jnp.ones((LANES,), jnp.float32)

    pltpu.sync_copy(labels_hbm.at[pl.ds(base, ROWS_PER_W)], labels_v)

    def zero_body(i, c):
        for u in range(8):
            a_v[pl.ds(8 * LANES * i + LANES * u, LANES)] = zeros
        return c
    lax.fori_loop(0, NL * CHUNKS // 8, zero_body, 0)
    for l in range(NL):
        q_v[pl.ds(LANES * l, LANES)] = zeros
        c_v[pl.ds(LANES * l, LANES)] = zeros

    bufs = (zbuf0, zbuf1)
    sems = (sem0, sem1)

    def start(k, row):
        pltpu.make_async_copy(z_hbm.at[base + row], bufs[k], sems[k]).start()

    def wait(k):
        pltpu.make_async_copy(z_hbm.at[base], bufs[k], sems[k]).wait()

    UNR = 16
    NACC = 8

    def process(buf, lab):
        off = lab * CH

        def body(j, qs):
            o = UNR * LANES * j
            zv = [buf[pl.ds(o + LANES * u, LANES)] for u in range(UNR)]
            for u in range(UNR):
                plsc.addupdate(a_v.at[pl.ds(off + o + LANES * u, LANES)],
                               zv[u])
            qs = list(qs)
            for u in range(UNR):
                qs[u % NACC] = qs[u % NACC] + zv[u] * zv[u]
            return tuple(qs)
        qs = lax.fori_loop(0, CHUNKS // UNR, body, (zeros,) * NACC)
        q = ((qs[0] + qs[1]) + (qs[2] + qs[3])) + \
            ((qs[4] + qs[5]) + (qs[6] + qs[7]))
        plsc.addupdate(q_v.at[pl.ds(lab * LANES, LANES)], q)
        plsc.addupdate(c_v.at[pl.ds(lab * LANES, LANES)], ones)

    NGROUPS = ROWS_PER_W // LANES
    start(0, 0)
    start(1, 1)

    def group_body(g, c):
        lv = labels_v[pl.ds(LANES * g, LANES)]
        for u in range(LANES):
            k = u % 2
            wait(k)
            process(bufs[k], lv[u])
            start(k, LANES * g + u + 2)
        return c
    lax.fori_loop(0, NGROUPS - 1, group_body, 0)
    lv = labels_v[pl.ds(LANES * (NGROUPS - 1), LANES)]
    for u in range(LANES):
        k = u % 2
        wait(k)
        process(bufs[k], lv[u])
        if u < LANES - 2:
            start(k, LANES * (NGROUPS - 1) + u + 2)

    pltpu.sync_copy(a_v, a_out.at[wid])
    pltpu.sync_copy(q_v, q_out.at[wid])
    pltpu.sync_copy(c_v, c_out.at[wid])


def _tc_partials_body(z_ref, lab_ref, a_ref, q_ref, c_ref):
    i = pl.program_id(0)
    z = z_ref[...]
    lab = lab_ref[0]
    oh = (lab[:, None] == lax.broadcasted_iota(jnp.int32, (1, LANES), 1)
          ).astype(jnp.float32)
    a = lax.dot_general(oh, z, (((0,), (0,)), ((), ())),
                        preferred_element_type=jnp.float32)
    rs = jnp.sum(z * z, axis=1, keepdims=True)
    ql = jnp.sum(rs * oh, axis=0, keepdims=True)
    cl = jnp.sum(oh, axis=0, keepdims=True)

    @pl.when(i == 0)
    def _():
        a_ref[...] = jnp.zeros_like(a_ref)
        q_ref[...] = jnp.zeros_like(q_ref)
        c_ref[...] = jnp.zeros_like(c_ref)
    a_ref[...] += a
    q_ref[...] += jnp.broadcast_to(ql, (8, LANES))
    c_ref[...] += jnp.broadcast_to(cl, (8, LANES))


def _finalize_body(a_sc, q_sc, c_sc, a_tc, q_tc, c_tc, out_ref):
    a = jnp.sum(a_sc[...], axis=0) + a_tc[...][:NL, :]
    q = jnp.sum(q_sc[...], axis=(0, 2)) + q_tc[...][0, :NL]
    n = jnp.sum(c_sc[...][:, :, 0], axis=0) + c_tc[...][0, :NL]
    ssq = jnp.sum(a * a, axis=1)
    safe = jnp.maximum(n, 1.0)
    chf = jnp.float32(CH)
    eps2 = jnp.float32(1e-16)
    mse = q / (safe * chf) - ssq / (safe * safe * chf) + eps2
    out_ref[...] = jnp.sum(jnp.where(n > 0, mse, 0.0)).reshape(1, 1)


@jax.jit
def _run(z3, labels):
    mesh = plsc.VectorSubcoreMesh(core_axis_name="c", subcore_axis_name="s")
    sc = pl.kernel(
        _sc_partials_kernel,
        mesh=mesh,
        out_type=(
            jax.ShapeDtypeStruct((NW, NL * CH), jnp.float32),
            jax.ShapeDtypeStruct((NW, NL * LANES), jnp.float32),
            jax.ShapeDtypeStruct((NW, NL * LANES), jnp.float32),
        ),
        scratch_types=[
            pltpu.VMEM((NL * CH,), jnp.float32),
            pltpu.VMEM((CH,), jnp.float32),
            pltpu.VMEM((CH,), jnp.float32),
            pltpu.VMEM((ROWS_PER_W,), jnp.int32),
            pltpu.VMEM((NL * LANES,), jnp.float32),
            pltpu.VMEM((NL * LANES,), jnp.float32),
            pltpu.SemaphoreType.DMA,
            pltpu.SemaphoreType.DMA,
        ],
    )
    z2d = z3.reshape(B, CH)
    a_sc, q_sc, c_sc = sc(z2d, labels)

    a_tc, q_tc, c_tc = pl.pallas_call(
        _tc_partials_body,
        grid=(NBLK_TC,),
        in_specs=[
            pl.BlockSpec((BB, CH), lambda i: (B_SC // BB + i, 0)),
            pl.BlockSpec((1, BB), lambda i: (0, B_SC // BB + i)),
        ],
        out_specs=[
            pl.BlockSpec((LANES, CH), lambda i: (0, 0)),
            pl.BlockSpec((8, LANES), lambda i: (0, 0)),
            pl.BlockSpec((8, LANES), lambda i: (0, 0)),
        ],
        out_shape=[
            jax.ShapeDtypeStruct((LANES, CH), jnp.float32),
            jax.ShapeDtypeStruct((8, LANES), jnp.float32),
            jax.ShapeDtypeStruct((8, LANES), jnp.float32),
        ],
    )(z2d, labels.reshape(1, B))

    out = pl.pallas_call(
        _finalize_body,
        out_shape=jax.ShapeDtypeStruct((1, 1), jnp.float32),
    )(a_sc.reshape(NW, NL, CH),
      q_sc.reshape(NW, NL, LANES),
      c_sc.reshape(NW, NL, LANES),
      a_tc, q_tc, c_tc)
    return out[0, 0]


def kernel(z, labels):
    return _run(z, labels)
